# Initial kernel scaffold; baseline (speedup 1.0000x reference)
import jax
import jax.numpy as jnp
from jax.experimental import pallas as pl


def kernel(pos, params, z, batch, edge_i, edge_j, idx_i, idx_j, idx_k, idx_kj, idx_ji):
    raise NotImplementedError("write your pallas kernel here")


# speedup vs baseline: 1.0537x; 1.0537x over previous
import functools

import jax
import jax.numpy as jnp
from jax.experimental import pallas as pl

N = 10000
E = 160000
T = 480000
H = 128
NR = 6
NS = 7
NB = 8
OUT = 128
NL = 3
NBLK = 2
G = 256
ZD = 2
XD = 66
DH = 256
CUT = 5.0
P_EXP = 5

BT = 1920
_INTERPRET = False


def _swish(x):
    return x * jax.nn.sigmoid(x)


def _envelope(x):
    p = P_EXP
    a = -(p + 1) * (p + 2) / 2.0
    b = p * (p + 2)
    c = -p * (p + 1) / 2.0
    env = 1.0 / x + a * x ** (p - 1) + b * x ** p + c * x ** (p + 1)
    return jnp.where(x < 1.0, env, 0.0)


def _bilinear_body(s2_ref, xg_ref, w_ref, out_ref):
    xg = xg_ref[...]
    s2 = s2_ref[...]
    y1 = jnp.concatenate([s2[:, j:j + 1] * xg for j in range(NB)], axis=1)
    out_ref[...] = jax.lax.dot_general(
        y1.astype(jnp.bfloat16), w_ref[...],
        (((1,), (0,)), ((), ())),
        preferred_element_type=jnp.float32)


@functools.partial(jax.jit, static_argnames=())
def _bilinear(s2, xg, w2):
    return pl.pallas_call(
        _bilinear_body,
        grid=(T // BT,),
        in_specs=[
            pl.BlockSpec((BT, NB), lambda i: (i, 0)),
            pl.BlockSpec((BT, H), lambda i: (i, 0)),
            pl.BlockSpec((NB * H, H), lambda i: (0, 0)),
        ],
        out_specs=pl.BlockSpec((BT, H), lambda i: (i, 0)),
        out_shape=jax.ShapeDtypeStruct((T, H), jnp.float32),
        interpret=_INTERPRET,
    )(s2, xg, w2)


def kernel(pos, params, z, batch, edge_i, edge_j, idx_i, idx_j, idx_k, idx_kj, idx_ji):
    dist = jnp.sqrt(jnp.sum((pos[edge_i] - pos[edge_j]) ** 2, axis=-1) + 1e-9)
    pos_i = pos[idx_i]
    pji = pos[idx_j] - pos_i
    pki = pos[idx_k] - pos_i
    a = jnp.sum(pji * pki, axis=-1)
    b = jnp.linalg.norm(jnp.cross(pji, pki), axis=-1)
    angle = jnp.arctan2(b, a)
    d = dist / CUT
    rbf = _envelope(d)[:, None] * jnp.sin(params['freq'][None, :] * d[:, None])
    dk = d[idx_kj]
    nrange = jnp.arange(1, NR + 1, dtype=jnp.float32)
    radial = _envelope(dk)[:, None] * jnp.sin(jnp.pi * nrange[None, :] * dk[:, None])
    lr = jnp.arange(NS, dtype=jnp.float32)
    ang = jnp.cos(lr[None, :] * angle[:, None])
    sbf = (ang[:, :, None] * radial[:, None, :]).reshape(T, NS * NR)

    h = params['emb_table'][z]
    rbf_e = _swish(rbf @ params['emb_rbf_W'])
    x = _swish(jnp.concatenate([h[edge_j], h[edge_i], rbf_e], axis=-1)
               @ params['emb_W'] + params['emb_b'])

    def outblock(k_, x_):
        t = (rbf @ params['o%d_rbf' % k_]) * x_
        t = jax.ops.segment_sum(t, edge_i, num_segments=N)
        for li in range(NL):
            t = _swish(t @ params['o%d_lW' % k_][li] + params['o%d_lb' % k_][li])
        return t @ params['o%d_out' % k_]

    P = outblock(0, x)
    for k_ in range(NBLK):
        r2 = rbf @ params['i%d_rbf' % k_]
        s2 = sbf @ params['i%d_sbf' % k_]
        x_ji = _swish(x @ params['i%d_jiW' % k_] + params['i%d_jib' % k_])
        x_kj = _swish(x @ params['i%d_kjW' % k_] + params['i%d_kjb' % k_]) * r2
        w3 = jnp.transpose(params['i%d_bil' % k_], (1, 2, 0)).reshape(NB * H, H)
        t = _bilinear(s2, x_kj[idx_kj], w3.astype(jnp.bfloat16))
        x_kj = jax.ops.segment_sum(t, idx_ji, num_segments=E)
        x = _swish((x_ji + x_kj) @ params['i%d_lW' % k_] + params['i%d_lb' % k_])
        P = P + outblock(k_ + 1, x)

    Pg = jax.ops.segment_sum(P, batch, num_segments=G)
    hh = Pg
    for li in range(4):
        hh = jnp.tanh(hh @ params['enc_W'][li] + params['enc_b'][li])
    mu = hh @ params['encmu_W'] + params['encmu_b']
    lv = hh @ params['enclv_W'] + params['enclv_b']
    eps = jax.random.normal(jax.random.key(1), (G, ZD), dtype=jnp.float32)
    zl = mu + jnp.exp(0.5 * lv) * eps
    hd = jnp.tanh(zl @ params['dec0_W'] + params['dec0_b'])
    for li in range(3):
        hd = jnp.tanh(hd @ params['dec_W'][li] + params['dec_b'][li])
    mud = hd @ params['decmu_W'] + params['decmu_b']
    lvd = hd @ params['declv_W'] + params['declv_b']
    return (mud, lvd, mu, lv)
